# Initial kernel scaffold; baseline (speedup 1.0000x reference)
import jax
import jax.numpy as jnp
from jax.experimental import pallas as pl


def kernel(edge_index, user_w, item_w):
    raise NotImplementedError("write your pallas kernel here")


# speedup vs baseline: 7.8331x; 7.8331x over previous
import functools

import jax
import jax.numpy as jnp
from jax import lax
from jax.experimental import pallas as pl
from jax.experimental.pallas import tpu as pltpu
from jax.experimental.pallas import tpu_sc as plsc

N_NODES = 50000
HALF = 25000
EMB = 64
E = 800000
K = 128
NCHUNKS = E // K
NC = 2
NS = 16
ACC_ROWS = 25600
ZCH = 200
N_ZCH = ACC_ROWS // ZCH
N_WCH = HALF // ZCH
DEG_CH = 1000
N_DEG_CH = HALF // DEG_CH

_mesh = plsc.VectorSubcoreMesh(core_axis_name="c", subcore_axis_name="s")
_sc_params = pltpu.CompilerParams(use_tc_tiling_on_sc=False)


@functools.partial(
    pl.kernel,
    mesh=_mesh,
    out_type=(
        jax.ShapeDtypeStruct((N_NODES,), jnp.float32),
        jax.ShapeDtypeStruct((NC, NCHUNKS, K), jnp.int32),
    ),
    scratch_types=[
        pltpu.VMEM_SHARED((ACC_ROWS,), jnp.float32),
        pltpu.VMEM((K,), jnp.int32),
        pltpu.VMEM((K,), jnp.int32),
        pltpu.VMEM((K,), jnp.float32),
        pltpu.VMEM((ACC_ROWS // NS,), jnp.float32),
        pltpu.VMEM((DEG_CH,), jnp.float32),
        pltpu.SemaphoreType.DMA,
    ],
    compiler_params=_sc_params,
)
def _sc_prep(row_hbm, deg_out, lidx_out, deg_acc, row_v, lidx_v, ones_v,
             zero_v, dstage, sem):
    cid = lax.axis_index("c")
    sid = lax.axis_index("s")
    zrows = ACC_ROWS // NS

    def _z(i, _):
        zero_v[pl.ds(i * 16, 16)] = jnp.zeros((16,), jnp.float32)
        return 0
    lax.fori_loop(0, zrows // 16, _z, 0)
    for t in range(K // 16):
        ones_v[pl.ds(t * 16, 16)] = jnp.ones((16,), jnp.float32)
    pltpu.sync_copy(zero_v, deg_acc.at[pl.ds(sid * zrows, zrows)])
    plsc.subcore_barrier()

    nch = 390 + jnp.where(sid < NCHUNKS - 390 * NS, 1, 0)

    def _chunk(i, _):
        j = i * NS + sid
        base = pl.multiple_of(j * K, K)
        pltpu.sync_copy(row_hbm.at[pl.ds(base, K)], row_v)
        for t in range(K // 16):
            r16 = row_v[pl.ds(t * 16, 16)]
            lr = r16 - cid * HALF
            ok = (lr >= 0) & (lr < HALF)
            dummy = HALF + t * 16 + lax.iota(jnp.int32, 16)
            lidx_v[pl.ds(t * 16, 16)] = jnp.where(ok, lr, dummy)
        pltpu.sync_copy(lidx_v, lidx_out.at[cid, j])
        pltpu.async_copy(ones_v, deg_acc.at[lidx_v], sem, add=True).wait()
        return 0

    lax.fori_loop(0, nch, _chunk, 0)
    plsc.subcore_barrier()

    for i in range((N_DEG_CH + NS - 1) // NS):
        j = i * NS + sid
        @pl.when(j < N_DEG_CH)
        def _():
            pltpu.sync_copy(deg_acc.at[pl.ds(j * DEG_CH, DEG_CH)], dstage)
            pltpu.sync_copy(
                dstage, deg_out.at[pl.ds(cid * HALF + j * DEG_CH, DEG_CH)])


@functools.partial(
    pl.kernel,
    mesh=_mesh,
    out_type=jax.ShapeDtypeStruct((N_NODES, EMB), jnp.float32),
    scratch_types=[
        pltpu.VMEM_SHARED((ACC_ROWS, EMB), jnp.float32),
        pltpu.VMEM((K,), jnp.int32),
        pltpu.VMEM((1, K), jnp.int32),
        pltpu.VMEM((K, EMB), jnp.float32),
        pltpu.VMEM((ZCH, EMB), jnp.float32),
        pltpu.SemaphoreType.DMA,
        pltpu.SemaphoreType.DMA,
    ],
    compiler_params=_sc_params,
)
def _sc_layer(w_hbm, col_hbm, lidx_hbm, zeros_hbm, acc_out, acc, col_v,
              lidx_v, rows_v, wstage, gsem, ssem):
    cid = lax.axis_index("c")
    sid = lax.axis_index("s")

    for i in range(N_ZCH // NS):
        j = i * NS + sid
        pltpu.sync_copy(zeros_hbm, acc.at[pl.ds(j * ZCH, ZCH), :])
    plsc.subcore_barrier()

    nch = 390 + jnp.where(sid < NCHUNKS - 390 * NS, 1, 0)

    def _chunk(i, _):
        j = i * NS + sid
        base = pl.multiple_of(j * K, K)
        pltpu.sync_copy(col_hbm.at[pl.ds(base, K)], col_v)
        pltpu.sync_copy(lidx_hbm.at[cid, pl.ds(j, 1)], lidx_v)
        pltpu.async_copy(w_hbm.at[col_v], rows_v, gsem).wait()
        pltpu.async_copy(rows_v, acc.at[lidx_v.at[0]], ssem, add=True).wait()
        return 0

    lax.fori_loop(0, nch, _chunk, 0)
    plsc.subcore_barrier()

    for i in range((N_WCH + NS - 1) // NS):
        j = i * NS + sid
        @pl.when(j < N_WCH)
        def _():
            pltpu.sync_copy(acc.at[pl.ds(j * ZCH, ZCH), :], wstage)
            pltpu.sync_copy(
                wstage, acc_out.at[pl.ds(cid * HALF + j * ZCH, ZCH), :])


_BS = 1000
_GRID = N_NODES // _BS


def _dinv(d):
    return jnp.where(d > 0, lax.rsqrt(d), 0.0)


def _scale_body(deg_ref, emb_ref, w_ref):
    w_ref[...] = emb_ref[...] * _dinv(deg_ref[...])


def _update_body(deg_ref, emb_ref, acc_ref, emb_n_ref, w_n_ref):
    di = _dinv(deg_ref[...])
    e = emb_ref[...] + di * acc_ref[...]
    emb_n_ref[...] = e
    w_n_ref[...] = e * di


def _final_body(deg_ref, emb_ref, acc_ref, emb_n_ref):
    emb_n_ref[...] = emb_ref[...] + _dinv(deg_ref[...]) * acc_ref[...]


_deg_spec = pl.BlockSpec((_BS, 1), lambda i: (i, 0))
_emb_spec = pl.BlockSpec((_BS, EMB), lambda i: (i, 0))
_emb_out = jax.ShapeDtypeStruct((N_NODES, EMB), jnp.float32)

_tc_scale = pl.pallas_call(
    _scale_body, grid=(_GRID,), in_specs=[_deg_spec, _emb_spec],
    out_specs=_emb_spec, out_shape=_emb_out)
_tc_update = pl.pallas_call(
    _update_body, grid=(_GRID,), in_specs=[_deg_spec, _emb_spec, _emb_spec],
    out_specs=(_emb_spec, _emb_spec), out_shape=(_emb_out, _emb_out))
_tc_final = pl.pallas_call(
    _final_body, grid=(_GRID,), in_specs=[_deg_spec, _emb_spec, _emb_spec],
    out_specs=_emb_spec, out_shape=_emb_out)


def kernel(edge_index, user_w, item_w):
    ei = edge_index.astype(jnp.int32)
    row, col = ei[0], ei[1]
    emb0 = jnp.concatenate([user_w, item_w], axis=0)

    deg, lidx = _sc_prep(row)
    deg2 = deg.reshape(N_NODES, 1)
    zeros = jnp.zeros((ZCH, EMB), jnp.float32)

    w0 = _tc_scale(deg2, emb0)
    acc1 = _sc_layer(w0, col, lidx, zeros)
    emb1, w1 = _tc_update(deg2, emb0, acc1)
    acc2 = _sc_layer(w1, col, lidx, zeros)
    emb2 = _tc_final(deg2, emb1, acc2)

    return emb2[:HALF], emb2[HALF:]


# speedup vs baseline: 10.7207x; 1.3686x over previous
import functools

import jax
import jax.numpy as jnp
from jax import lax
from jax.experimental import pallas as pl
from jax.experimental.pallas import tpu as pltpu
from jax.experimental.pallas import tpu_sc as plsc

N_NODES = 50000
HALF = 25000
EMB = 64
E = 800000
K = 128
NC = 2
NS = 16
CPT = 396
EPT = CPT * K
E_PAD = NS * EPT
NCHUNKS = NS * CPT
NROW = 3
NIDX = 6
RP = 6
GP = CPT // RP
ACC_ROWS = 25216
ZCH = 128
N_ZCH = ACC_ROWS // ZCH
N_WCH = HALF // ZCH
WREM = HALF - N_WCH * ZCH
DEG_CH = 1000
N_DEG_CH = HALF // DEG_CH
ROW_PAD = 60000

_mesh = plsc.VectorSubcoreMesh(core_axis_name="c", subcore_axis_name="s")
_sc_params = pltpu.CompilerParams(use_tc_tiling_on_sc=False)


@functools.partial(
    pl.kernel,
    mesh=_mesh,
    out_type=(
        jax.ShapeDtypeStruct((N_NODES,), jnp.float32),
        jax.ShapeDtypeStruct((NC, NCHUNKS, 2, K), jnp.int32),
    ),
    scratch_types=[
        pltpu.VMEM_SHARED((ACC_ROWS,), jnp.float32),
        pltpu.VMEM((2, RP * K), jnp.int32),
        pltpu.VMEM((2, RP * K), jnp.int32),
        pltpu.VMEM((RP, 2, K), jnp.int32),
        pltpu.VMEM((RP, 2, K), jnp.int32),
        pltpu.VMEM((K,), jnp.float32),
        pltpu.VMEM((1584,), jnp.float32),
        pltpu.VMEM((DEG_CH,), jnp.float32),
        pltpu.SemaphoreType.DMA,
        pltpu.SemaphoreType.DMA,
        pltpu.SemaphoreType.DMA,
        pltpu.SemaphoreType.DMA,
    ],
    compiler_params=_sc_params,
)
def _sc_prep(rc_hbm, deg_out, pk_out, deg_acc, rc0, rc1, pk0, pk1,
             ones_v, zero_v, dstage, isem0, isem1, wsem, ssem):
    cid = lax.axis_index("c")
    sid = lax.axis_index("s")
    tb = sid * EPT
    cb0 = sid * CPT
    rcb = (rc0, rc1)
    pkb = (pk0, pk1)
    isem = (isem0, isem1)
    CB = RP * K

    def _z(i, _):
        zero_v[pl.ds(i * 16, 16)] = jnp.zeros((16,), jnp.float32)
        return 0
    lax.fori_loop(0, 99, _z, 0)
    for t in range(K // 16):
        ones_v[pl.ds(t * 16, 16)] = jnp.ones((16,), jnp.float32)
    pltpu.sync_copy(zero_v.at[pl.ds(0, 1576)],
                    deg_acc.at[pl.ds(sid * 1576, 1576)])
    plsc.subcore_barrier()

    def _rc_load(p, g):
        return pltpu.make_async_copy(
            rc_hbm.at[:, pl.ds(tb + g * CB, CB)], rcb[p], isem[p])

    def _pk_write(p, g):
        return pltpu.make_async_copy(
            pkb[p], pk_out.at[cid, pl.ds(cb0 + g * RP, RP)], wsem)

    def _scatter(p, b):
        return pltpu.make_async_copy(
            ones_v, deg_acc.at[pkb[p].at[b, 1]], ssem)

    def _batch(p, g):
        _rc_load(p, g).wait()
        @pl.when(g + 1 < GP)
        def _():
            pltpu.async_copy(
                rc_hbm.at[:, pl.ds(tb + (g + 1) * CB, CB)], rcb[p ^ 1],
                isem[p ^ 1])
        @pl.when(g >= 1)
        def _():
            _pk_write(p ^ 1, g).wait()
            for b in range(RP):
                _scatter(p ^ 1, b).wait()
        for b in range(RP):
            for t in range(K // 16):
                c16 = rcb[p][1, pl.ds(b * K + t * 16, 16)]
                pkb[p][b, 0, pl.ds(t * 16, 16)] = c16
                r16 = rcb[p][0, pl.ds(b * K + t * 16, 16)]
                lr = r16 - cid * HALF
                ok = (lr >= 0) & (lr < HALF)
                dummy = HALF + t * 16 + lax.iota(jnp.int32, 16)
                pkb[p][b, 1, pl.ds(t * 16, 16)] = jnp.where(ok, lr, dummy)
        pltpu.async_copy(pkb[p], pk_out.at[cid, pl.ds(cb0 + g * RP, RP)],
                         wsem)
        for b in range(RP):
            pltpu.async_copy(ones_v, deg_acc.at[pkb[p].at[b, 1]], ssem,
                             add=True)

    pltpu.async_copy(rc_hbm.at[:, pl.ds(tb, CB)], rc0, isem0)

    def _pair(i, _):
        _batch(0, 2 * i)
        _batch(1, 2 * i + 1)
        return 0
    lax.fori_loop(0, GP // 2, _pair, 0)

    _pk_write(1, GP - 1).wait()
    for b in range(RP):
        _scatter(1, b).wait()
    plsc.subcore_barrier()

    for i in range((N_DEG_CH + NS - 1) // NS):
        j = i * NS + sid
        @pl.when(j < N_DEG_CH)
        def _():
            pltpu.sync_copy(deg_acc.at[pl.ds(j * DEG_CH, DEG_CH)], dstage)
            pltpu.sync_copy(
                dstage, deg_out.at[pl.ds(cid * HALF + j * DEG_CH, DEG_CH)])


@functools.partial(
    pl.kernel,
    mesh=_mesh,
    out_type=jax.ShapeDtypeStruct((N_NODES, EMB), jnp.float32),
    scratch_types=(
        [pltpu.VMEM_SHARED((ACC_ROWS, EMB), jnp.float32)]
        + [pltpu.VMEM((K, EMB), jnp.float32) for _ in range(NROW)]
        + [pltpu.VMEM((2, K), jnp.int32) for _ in range(NIDX)]
        + [pltpu.SemaphoreType.DMA for _ in range(NIDX + 2 * NROW)]
    ),
    compiler_params=_sc_params,
)
def _sc_layer(w_hbm, pk_hbm, zeros_hbm, acc_out, acc, *bufs):
    rows = bufs[:NROW]
    cidx = bufs[NROW:NROW + NIDX]
    isem = bufs[NROW + NIDX:NROW + 2 * NIDX]
    gsem = bufs[NROW + 2 * NIDX:NROW + 2 * NIDX + NROW]
    ssem = bufs[NROW + 2 * NIDX + NROW:]
    cid = lax.axis_index("c")
    sid = lax.axis_index("s")
    cb0 = sid * CPT

    for i in range((N_ZCH + NS - 1) // NS):
        j = i * NS + sid
        @pl.when(j < N_ZCH)
        def _():
            pltpu.sync_copy(zeros_hbm, acc.at[pl.ds(j * ZCH, ZCH), :])
    plsc.subcore_barrier()

    def _idx_load(c, q):
        pltpu.async_copy(pk_hbm.at[cid, cb0 + c], cidx[q], isem[q])

    def _chunk(c, u, first_in_ring, has_prev):
        q3 = u % NROW
        q6 = u % NIDX
        if not first_in_ring:
            pltpu.make_async_copy(rows[q3], acc.at[cidx[q6].at[1]],
                                  ssem[q3]).wait()
        pltpu.make_async_copy(pk_hbm.at[cid, cb0 + c], cidx[q6],
                              isem[q6]).wait()
        pltpu.async_copy(w_hbm.at[cidx[q6].at[0]], rows[q3], gsem[q3])
        @pl.when(c + 2 < CPT)
        def _():
            _idx_load(c + 2, (u + 2) % NIDX)
        if has_prev:
            pq3 = (u - 1) % NROW
            pq6 = (u - 1) % NIDX
            pltpu.make_async_copy(w_hbm.at[cidx[pq6].at[0]], rows[pq3],
                                  gsem[pq3]).wait()
            pltpu.async_copy(rows[pq3], acc.at[cidx[pq6].at[1]], ssem[pq3],
                             add=True)

    _idx_load(0, 0)
    _idx_load(1, 1)
    for u in range(NIDX):
        _chunk(u, u, first_in_ring=(u < NROW), has_prev=(u >= 1))

    def _body(i, _):
        c0 = i * NIDX
        for u in range(NIDX):
            _chunk(c0 + u, u, first_in_ring=False, has_prev=True)
        return 0
    lax.fori_loop(1, CPT // NIDX, _body, 0)

    lu = (CPT - 1) % NIDX
    lq = (CPT - 1) % NROW
    pltpu.make_async_copy(w_hbm.at[cidx[lu].at[0]], rows[lq],
                          gsem[lq]).wait()
    pltpu.async_copy(rows[lq], acc.at[cidx[lu].at[1]], ssem[lq], add=True)
    for d in range(NROW):
        c = CPT - 1 - d
        pltpu.make_async_copy(rows[c % NROW], acc.at[cidx[c % NIDX].at[1]],
                              ssem[c % NROW]).wait()
    plsc.subcore_barrier()

    for i in range((N_WCH + NS - 1) // NS):
        j = i * NS + sid
        @pl.when(j < N_WCH)
        def _():
            pltpu.sync_copy(acc.at[pl.ds(j * ZCH, ZCH), :], rows[0])
            pltpu.sync_copy(
                rows[0], acc_out.at[pl.ds(cid * HALF + j * ZCH, ZCH), :])
    @pl.when(sid == NS - 1)
    def _():
        pltpu.sync_copy(acc.at[pl.ds(N_WCH * ZCH, WREM), :],
                        rows[1].at[pl.ds(0, WREM), :])
        pltpu.sync_copy(rows[1].at[pl.ds(0, WREM), :],
                        acc_out.at[pl.ds(cid * HALF + N_WCH * ZCH, WREM), :])


_BS = 1000
_GRID = N_NODES // _BS


def _dinv(d):
    return jnp.where(d > 0, lax.rsqrt(d), 0.0)


def _scale_body(deg_ref, emb_ref, w_ref):
    w_ref[...] = emb_ref[...] * _dinv(deg_ref[...])


def _update_body(deg_ref, emb_ref, acc_ref, emb_n_ref, w_n_ref):
    di = _dinv(deg_ref[...])
    e = emb_ref[...] + di * acc_ref[...]
    emb_n_ref[...] = e
    w_n_ref[...] = e * di


def _final_body(deg_ref, emb_ref, acc_ref, emb_n_ref):
    emb_n_ref[...] = emb_ref[...] + _dinv(deg_ref[...]) * acc_ref[...]


_deg_spec = pl.BlockSpec((_BS, 1), lambda i: (i, 0))
_emb_spec = pl.BlockSpec((_BS, EMB), lambda i: (i, 0))
_emb_out = jax.ShapeDtypeStruct((N_NODES, EMB), jnp.float32)

_tc_scale = pl.pallas_call(
    _scale_body, grid=(_GRID,), in_specs=[_deg_spec, _emb_spec],
    out_specs=_emb_spec, out_shape=_emb_out)
_tc_update = pl.pallas_call(
    _update_body, grid=(_GRID,), in_specs=[_deg_spec, _emb_spec, _emb_spec],
    out_specs=(_emb_spec, _emb_spec), out_shape=(_emb_out, _emb_out))
_tc_final = pl.pallas_call(
    _final_body, grid=(_GRID,), in_specs=[_deg_spec, _emb_spec, _emb_spec],
    out_specs=_emb_spec, out_shape=_emb_out)


def kernel(edge_index, user_w, item_w):
    ei = edge_index.astype(jnp.int32)
    npad = E_PAD - E
    row = jnp.concatenate([ei[0], jnp.full((npad,), ROW_PAD, jnp.int32)])
    col = jnp.concatenate([ei[1], jnp.zeros((npad,), jnp.int32)])
    rc = jnp.stack([row, col])
    emb0 = jnp.concatenate([user_w, item_w], axis=0)

    deg, pk = _sc_prep(rc)
    deg2 = deg.reshape(N_NODES, 1)
    zeros = jnp.zeros((ZCH, EMB), jnp.float32)

    w0 = _tc_scale(deg2, emb0)
    acc1 = _sc_layer(w0, pk, zeros)
    emb1, w1 = _tc_update(deg2, emb0, acc1)
    acc2 = _sc_layer(w1, pk, zeros)
    emb2 = _tc_final(deg2, emb1, acc2)

    return emb2[:HALF], emb2[HALF:]


# speedup vs baseline: 10.7398x; 1.0018x over previous
import functools

import jax
import jax.numpy as jnp
from jax import lax
from jax.experimental import pallas as pl
from jax.experimental.pallas import tpu as pltpu
from jax.experimental.pallas import tpu_sc as plsc

N_NODES = 50000
HALF = 25000
EMB = 64
E = 800000
K = 128
NC = 2
NS = 16
CPT = 396
EPT = CPT * K
E_PAD = NS * EPT
NCHUNKS = NS * CPT
NROW = 3
NIDX = 6
RP = 6
GP = CPT // RP
ACC_ROWS = 25216
ZCH = 128
N_ZCH = ACC_ROWS // ZCH
N_WCH = HALF // ZCH
WREM = HALF - N_WCH * ZCH
DEG_CH = 1000
N_DEG_CH = HALF // DEG_CH
ROW_PAD = 60000

_mesh = plsc.VectorSubcoreMesh(core_axis_name="c", subcore_axis_name="s")
_sc_params = pltpu.CompilerParams(use_tc_tiling_on_sc=False)


@functools.partial(
    pl.kernel,
    mesh=_mesh,
    out_type=(
        jax.ShapeDtypeStruct((N_NODES,), jnp.float32),
        jax.ShapeDtypeStruct((NC, NCHUNKS, 2, K), jnp.int32),
    ),
    scratch_types=[
        pltpu.VMEM_SHARED((ACC_ROWS,), jnp.float32),
        pltpu.VMEM((2, RP * K), jnp.int32),
        pltpu.VMEM((2, RP * K), jnp.int32),
        pltpu.VMEM((RP, 2, K), jnp.int32),
        pltpu.VMEM((RP, 2, K), jnp.int32),
        pltpu.VMEM((K,), jnp.float32),
        pltpu.VMEM((1584,), jnp.float32),
        pltpu.VMEM((DEG_CH,), jnp.float32),
        pltpu.SemaphoreType.DMA,
        pltpu.SemaphoreType.DMA,
        pltpu.SemaphoreType.DMA,
        pltpu.SemaphoreType.DMA,
    ],
    compiler_params=_sc_params,
)
def _sc_prep(rc_hbm, deg_out, pk_out, deg_acc, rc0, rc1, pk0, pk1,
             ones_v, zero_v, dstage, isem0, isem1, wsem, ssem):
    cid = lax.axis_index("c")
    sid = lax.axis_index("s")
    tb = sid * EPT
    cb0 = sid * CPT
    rcb = (rc0, rc1)
    pkb = (pk0, pk1)
    isem = (isem0, isem1)
    CB = RP * K

    def _z(i, _):
        zero_v[pl.ds(i * 16, 16)] = jnp.zeros((16,), jnp.float32)
        return 0
    lax.fori_loop(0, 99, _z, 0)
    for t in range(K // 16):
        ones_v[pl.ds(t * 16, 16)] = jnp.ones((16,), jnp.float32)
    pltpu.sync_copy(zero_v.at[pl.ds(0, 1576)],
                    deg_acc.at[pl.ds(sid * 1576, 1576)])
    plsc.subcore_barrier()

    def _rc_load(p, g):
        return pltpu.make_async_copy(
            rc_hbm.at[:, pl.ds(tb + g * CB, CB)], rcb[p], isem[p])

    def _pk_write(p, g):
        return pltpu.make_async_copy(
            pkb[p], pk_out.at[cid, pl.ds(cb0 + g * RP, RP)], wsem)

    def _scatter(p, b):
        return pltpu.make_async_copy(
            ones_v, deg_acc.at[pkb[p].at[b, 1]], ssem)

    def _batch(p, g):
        _rc_load(p, g).wait()
        @pl.when(g + 1 < GP)
        def _():
            pltpu.async_copy(
                rc_hbm.at[:, pl.ds(tb + (g + 1) * CB, CB)], rcb[p ^ 1],
                isem[p ^ 1])
        @pl.when(g >= 1)
        def _():
            _pk_write(p ^ 1, g).wait()
            for b in range(RP):
                _scatter(p ^ 1, b).wait()
        for b in range(RP):
            for t in range(K // 16):
                c16 = rcb[p][1, pl.ds(b * K + t * 16, 16)]
                pkb[p][b, 0, pl.ds(t * 16, 16)] = c16
                r16 = rcb[p][0, pl.ds(b * K + t * 16, 16)]
                lr = r16 - cid * HALF
                ok = (lr >= 0) & (lr < HALF)
                dummy = (HALF + sid * 13
                         + (t * 16 + lax.iota(jnp.int32, 16)) % 13)
                pkb[p][b, 1, pl.ds(t * 16, 16)] = jnp.where(ok, lr, dummy)
        pltpu.async_copy(pkb[p], pk_out.at[cid, pl.ds(cb0 + g * RP, RP)],
                         wsem)
        for b in range(RP):
            pltpu.async_copy(ones_v, deg_acc.at[pkb[p].at[b, 1]], ssem,
                             add=True)

    pltpu.async_copy(rc_hbm.at[:, pl.ds(tb, CB)], rc0, isem0)

    def _pair(i, _):
        _batch(0, 2 * i)
        _batch(1, 2 * i + 1)
        return 0
    lax.fori_loop(0, GP // 2, _pair, 0)

    _pk_write(1, GP - 1).wait()
    for b in range(RP):
        _scatter(1, b).wait()
    plsc.subcore_barrier()

    for i in range((N_DEG_CH + NS - 1) // NS):
        j = i * NS + sid
        @pl.when(j < N_DEG_CH)
        def _():
            pltpu.sync_copy(deg_acc.at[pl.ds(j * DEG_CH, DEG_CH)], dstage)
            pltpu.sync_copy(
                dstage, deg_out.at[pl.ds(cid * HALF + j * DEG_CH, DEG_CH)])


@functools.partial(
    pl.kernel,
    mesh=_mesh,
    out_type=jax.ShapeDtypeStruct((N_NODES, EMB), jnp.float32),
    scratch_types=(
        [pltpu.VMEM_SHARED((ACC_ROWS, EMB), jnp.float32)]
        + [pltpu.VMEM((K, EMB), jnp.float32) for _ in range(NROW)]
        + [pltpu.VMEM((2, K), jnp.int32) for _ in range(NIDX)]
        + [pltpu.SemaphoreType.DMA for _ in range(NIDX + 2 * NROW)]
    ),
    compiler_params=_sc_params,
)
def _sc_layer(w_hbm, pk_hbm, zeros_hbm, acc_out, acc, *bufs):
    rows = bufs[:NROW]
    cidx = bufs[NROW:NROW + NIDX]
    isem = bufs[NROW + NIDX:NROW + 2 * NIDX]
    gsem = bufs[NROW + 2 * NIDX:NROW + 2 * NIDX + NROW]
    ssem = bufs[NROW + 2 * NIDX + NROW:]
    cid = lax.axis_index("c")
    sid = lax.axis_index("s")
    cb0 = sid * CPT

    for i in range((N_ZCH + NS - 1) // NS):
        j = i * NS + sid
        @pl.when(j < N_ZCH)
        def _():
            pltpu.sync_copy(zeros_hbm, acc.at[pl.ds(j * ZCH, ZCH), :])
    plsc.subcore_barrier()

    def _idx_load(c, q):
        pltpu.async_copy(pk_hbm.at[cid, cb0 + c], cidx[q], isem[q])

    def _chunk(c, u, first_in_ring, has_prev):
        q3 = u % NROW
        q6 = u % NIDX
        if not first_in_ring:
            pltpu.make_async_copy(rows[q3], acc.at[cidx[q6].at[1]],
                                  ssem[q3]).wait()
        pltpu.make_async_copy(pk_hbm.at[cid, cb0 + c], cidx[q6],
                              isem[q6]).wait()
        pltpu.async_copy(w_hbm.at[cidx[q6].at[0]], rows[q3], gsem[q3])
        @pl.when(c + 2 < CPT)
        def _():
            _idx_load(c + 2, (u + 2) % NIDX)
        if has_prev:
            pq3 = (u - 1) % NROW
            pq6 = (u - 1) % NIDX
            pltpu.make_async_copy(w_hbm.at[cidx[pq6].at[0]], rows[pq3],
                                  gsem[pq3]).wait()
            pltpu.async_copy(rows[pq3], acc.at[cidx[pq6].at[1]], ssem[pq3],
                             add=True)

    _idx_load(0, 0)
    _idx_load(1, 1)
    for u in range(NIDX):
        _chunk(u, u, first_in_ring=(u < NROW), has_prev=(u >= 1))

    def _body(i, _):
        c0 = i * NIDX
        for u in range(NIDX):
            _chunk(c0 + u, u, first_in_ring=False, has_prev=True)
        return 0
    lax.fori_loop(1, CPT // NIDX, _body, 0)

    lu = (CPT - 1) % NIDX
    lq = (CPT - 1) % NROW
    pltpu.make_async_copy(w_hbm.at[cidx[lu].at[0]], rows[lq],
                          gsem[lq]).wait()
    pltpu.async_copy(rows[lq], acc.at[cidx[lu].at[1]], ssem[lq], add=True)
    for d in range(NROW):
        c = CPT - 1 - d
        pltpu.make_async_copy(rows[c % NROW], acc.at[cidx[c % NIDX].at[1]],
                              ssem[c % NROW]).wait()
    plsc.subcore_barrier()

    for i in range((N_WCH + NS - 1) // NS):
        j = i * NS + sid
        @pl.when(j < N_WCH)
        def _():
            pltpu.sync_copy(acc.at[pl.ds(j * ZCH, ZCH), :], rows[0])
            pltpu.sync_copy(
                rows[0], acc_out.at[pl.ds(cid * HALF + j * ZCH, ZCH), :])
    @pl.when(sid == NS - 1)
    def _():
        pltpu.sync_copy(acc.at[pl.ds(N_WCH * ZCH, WREM), :],
                        rows[1].at[pl.ds(0, WREM), :])
        pltpu.sync_copy(rows[1].at[pl.ds(0, WREM), :],
                        acc_out.at[pl.ds(cid * HALF + N_WCH * ZCH, WREM), :])


_BS = 1000
_GRID = N_NODES // _BS


def _dinv(d):
    return jnp.where(d > 0, lax.rsqrt(d), 0.0)


def _scale_body(deg_ref, emb_ref, w_ref):
    w_ref[...] = emb_ref[...] * _dinv(deg_ref[...])


def _update_body(deg_ref, emb_ref, acc_ref, emb_n_ref, w_n_ref):
    di = _dinv(deg_ref[...])
    e = emb_ref[...] + di * acc_ref[...]
    emb_n_ref[...] = e
    w_n_ref[...] = e * di


def _final_body(deg_ref, emb_ref, acc_ref, emb_n_ref):
    emb_n_ref[...] = emb_ref[...] + _dinv(deg_ref[...]) * acc_ref[...]


_deg_spec = pl.BlockSpec((_BS, 1), lambda i: (i, 0))
_emb_spec = pl.BlockSpec((_BS, EMB), lambda i: (i, 0))
_emb_out = jax.ShapeDtypeStruct((N_NODES, EMB), jnp.float32)

_tc_scale = pl.pallas_call(
    _scale_body, grid=(_GRID,), in_specs=[_deg_spec, _emb_spec],
    out_specs=_emb_spec, out_shape=_emb_out)
_tc_update = pl.pallas_call(
    _update_body, grid=(_GRID,), in_specs=[_deg_spec, _emb_spec, _emb_spec],
    out_specs=(_emb_spec, _emb_spec), out_shape=(_emb_out, _emb_out))
_tc_final = pl.pallas_call(
    _final_body, grid=(_GRID,), in_specs=[_deg_spec, _emb_spec, _emb_spec],
    out_specs=_emb_spec, out_shape=_emb_out)


def kernel(edge_index, user_w, item_w):
    ei = edge_index.astype(jnp.int32)
    npad = E_PAD - E
    row = jnp.concatenate([ei[0], jnp.full((npad,), ROW_PAD, jnp.int32)])
    col = jnp.concatenate([ei[1], jnp.zeros((npad,), jnp.int32)])
    rc = jnp.stack([row, col])
    emb0 = jnp.concatenate([user_w, item_w], axis=0)

    deg, pk = _sc_prep(rc)
    deg2 = deg.reshape(N_NODES, 1)
    zeros = jnp.zeros((ZCH, EMB), jnp.float32)

    w0 = _tc_scale(deg2, emb0)
    acc1 = _sc_layer(w0, pk, zeros)
    emb1, w1 = _tc_update(deg2, emb0, acc1)
    acc2 = _sc_layer(w1, pk, zeros)
    emb2 = _tc_final(deg2, emb1, acc2)

    return emb2[:HALF], emb2[HALF:]
